# baseline (speedup 1.0000x reference)
import functools

import jax
import jax.numpy as jnp
from jax.experimental import pallas as pl
from jax.experimental.pallas import tpu as pltpu

TOP_K = 8
N_EXPERTS = 64
BT = 1024


def _gate_kernel(x_ref, w_ref, idx_ref, out_w_ref):
    x = x_ref[...]
    w = w_ref[...]
    logits_t = jax.lax.dot_general(
        w, x, (((1,), (1,)), ((), ())),
        preferred_element_type=jnp.float32)
    m = jnp.max(logits_t, axis=0, keepdims=True)
    e = jnp.exp(logits_t - m)
    scores = e / jnp.sum(e, axis=0, keepdims=True)

    iota = jax.lax.broadcasted_iota(jnp.int32, scores.shape, 0)
    work = scores
    vals = []
    idxs = []
    for _ in range(TOP_K):
        mx = jnp.max(work, axis=0, keepdims=True)
        am = jnp.min(jnp.where(work == mx, iota, N_EXPERTS),
                     axis=0, keepdims=True)
        vals.append(mx)
        idxs.append(am)
        work = jnp.where(iota == am, -1.0, work)
    topv = jnp.concatenate(vals, axis=0)
    topi = jnp.concatenate(idxs, axis=0)
    denom = jnp.sum(topv, axis=0, keepdims=True) + 1e-20
    out_w_ref[...] = (topv / denom).T
    idx_ref[...] = topi.T


@functools.partial(jax.jit, static_argnames=("interpret",))
def kernel(hidden_states, weight, interpret=False):
    bsz, seq_len, h = hidden_states.shape
    n_tokens = bsz * seq_len
    x = hidden_states.reshape(n_tokens, h)

    grid = (n_tokens // BT,)
    topk_idx, topk_weight = pl.pallas_call(
        _gate_kernel,
        grid=grid,
        in_specs=[
            pl.BlockSpec((BT, h), lambda i: (i, 0)),
            pl.BlockSpec((N_EXPERTS, h), lambda i: (0, 0)),
        ],
        out_specs=[
            pl.BlockSpec((BT, TOP_K), lambda i: (i, 0)),
            pl.BlockSpec((BT, TOP_K), lambda i: (i, 0)),
        ],
        out_shape=[
            jax.ShapeDtypeStruct((n_tokens, TOP_K), jnp.int32),
            jax.ShapeDtypeStruct((n_tokens, TOP_K), jnp.float32),
        ],
        compiler_params=pltpu.CompilerParams(
            dimension_semantics=("parallel",),
        ),
        interpret=interpret,
    )(x, weight)
    return (topk_idx, topk_weight)
